# Initial kernel scaffold; baseline (speedup 1.0000x reference)
import jax
import jax.numpy as jnp
from jax.experimental import pallas as pl


def kernel(pred, target):
    raise NotImplementedError("write your pallas kernel here")


# speedup vs baseline: 1.6273x; 1.6273x over previous
import jax
import jax.numpy as jnp
from jax import lax
from jax.experimental import pallas as pl
from jax.experimental.pallas import tpu as pltpu

_NA = 5
_NC = 20
_H = 19
_W = 19
_NT = 50
_HW = _H * _W
_AW = (1.3221, 3.19275, 5.05587, 9.47112, 11.2364)
_AH = (1.73145, 4.00944, 8.09892, 4.84053, 10.0071)
_OBJECT_SCALE = 5.0
_NO_OBJECT_SCALE = 1.0
_SIL_THRESH = 0.6


def _iou_parts(x1, y1, w1, h1, x2, y2, w2, h2):
    mx = jnp.minimum(x1 - w1 / 2.0, x2 - w2 / 2.0)
    Mx = jnp.maximum(x1 + w1 / 2.0, x2 + w2 / 2.0)
    my = jnp.minimum(y1 - h1 / 2.0, y2 - h2 / 2.0)
    My = jnp.maximum(y1 + h1 / 2.0, y2 + h2 / 2.0)
    uw = Mx - mx
    uh = My - my
    cw = w1 + w2 - uw
    ch = h1 + h2 - uh
    mask = (cw > 0) & (ch > 0)
    inter = jnp.where(mask, cw * ch, 0.0)
    union = w1 * h1 + w2 * h2 - inter
    return inter / jnp.maximum(union, 1e-12)


def _body(pred_ref, tgt_ref, out_ref):
    b = pl.program_id(0)
    a = pl.program_id(1)

    p = pred_ref[0, 0]
    tgt = tgt_ref[0]

    pos = lax.broadcasted_iota(jnp.int32, (1, _HW), 1)
    ix = (pos % _W).astype(jnp.float32)
    jy = (pos // _W).astype(jnp.float32)

    sigx = jax.nn.sigmoid(p[0:1, :])
    sigy = jax.nn.sigmoid(p[1:2, :])
    tw = p[2:3, :]
    th = p[3:4, :]
    conf = jax.nn.sigmoid(p[4:5, :])
    cls = p[5:25, :]

    aw = jnp.float32(0.0)
    ah = jnp.float32(0.0)
    for k in range(_NA):
        sel_k = (a == k)
        aw = jnp.where(sel_k, jnp.float32(_AW[k]), aw)
        ah = jnp.where(sel_k, jnp.float32(_AH[k]), ah)

    bx = sigx + ix
    by = sigy + jy
    bw = jnp.exp(tw) * aw
    bh = jnp.exp(th) * ah

    tcls = tgt[:, 0:1]
    gxn = tgt[:, 1:2]
    valid = gxn > 0
    gx = gxn * _W
    gy = tgt[:, 2:3] * _H
    gw = tgt[:, 3:4] * _W
    gh = tgt[:, 4:5] * _H

    lane5 = lax.broadcasted_iota(jnp.int32, (_NT, _NA), 1)
    aw5 = jnp.zeros((_NT, _NA), jnp.float32)
    ah5 = jnp.zeros((_NT, _NA), jnp.float32)
    for k in range(_NA):
        sel_k = lane5 == k
        aw5 = jnp.where(sel_k, jnp.float32(_AW[k]), aw5)
        ah5 = jnp.where(sel_k, jnp.float32(_AH[k]), ah5)
    z = jnp.float32(0.0)
    anc_iou = _iou_parts(z, z, gw, gh, z, z, aw5, ah5)
    m5 = jnp.max(anc_iou, axis=1, keepdims=True)
    best = jnp.min(jnp.where(anc_iou == m5, lane5, _NA), axis=1, keepdims=True)

    gi = jnp.clip(jnp.floor(gx).astype(jnp.int32), 0, _W - 1)
    gj = jnp.clip(jnp.floor(gy).astype(jnp.int32), 0, _H - 1)
    gif = gi.astype(jnp.float32)
    gjf = gj.astype(jnp.float32)

    aw_b = jnp.zeros_like(gw)
    ah_b = jnp.zeros_like(gh)
    for k in range(_NA):
        sel_k = best == k
        aw_b = jnp.where(sel_k, jnp.float32(_AW[k]), aw_b)
        ah_b = jnp.where(sel_k, jnp.float32(_AH[k]), ah_b)

    tbx = gx - gif
    tby = gy - gjf
    tbw = jnp.log(jnp.maximum(gw, 1e-12) / aw_b)
    tbh = jnp.log(jnp.maximum(gh, 1e-12) / ah_b)

    pos_t = gj * _W + gi
    cellpos = lax.broadcasted_iota(jnp.int32, (_NT, _HW), 1)
    tio = lax.broadcasted_iota(jnp.int32, (_NT, _HW), 0)
    M = valid & (best == a) & (pos_t == cellpos)
    r = jnp.max(jnp.where(M, tio + 1, 0), axis=0, keepdims=True)
    owned = r > 0
    O = jnp.where(M & ((tio + 1) == r), 1.0, 0.0)

    ious = _iou_parts(gx, gy, gw, gh, bx, by, bw, bh)
    iou_v = jnp.where(valid, ious, 0.0)
    max_iou = jnp.max(iou_v, axis=0, keepdims=True)
    conf_base = jnp.where(max_iou > _SIL_THRESH, 0.0, _NO_OBJECT_SCALE)

    def sel(v):
        return jnp.sum(O * v, axis=0, keepdims=True)

    tb0 = jnp.where(owned, sel(tbx), 0.5)
    tb1 = jnp.where(owned, sel(tby), 0.5)
    tb2 = jnp.where(owned, sel(tbw), 0.0)
    tb3 = jnp.where(owned, sel(tbh), 0.0)
    tconf = jnp.where(owned, sel(ious), 0.0)
    kc = jnp.clip(sel(tcls).astype(jnp.int32), 0, _NC - 1)

    lcoord = (jnp.sum((sigx - tb0) ** 2) + jnp.sum((sigy - tb1) ** 2)
              + jnp.sum((tw - tb2) ** 2) + jnp.sum((th - tb3) ** 2))

    cm = jnp.where(owned, _OBJECT_SCALE, conf_base)
    lconf = jnp.sum(cm * (conf - tconf) ** 2)

    mC = jnp.max(cls, axis=0, keepdims=True)
    lse = jnp.log(jnp.sum(jnp.exp(cls - mC), axis=0, keepdims=True)) + mC
    kio = lax.broadcasted_iota(jnp.int32, (_NC, _HW), 0)
    logit_sel = jnp.sum(jnp.where(kio == kc, cls, 0.0), axis=0, keepdims=True)
    lcls = jnp.sum(jnp.where(owned, lse - logit_sel, 0.0))

    partial = 0.5 * (lcoord + lconf) + lcls

    @pl.when((b == 0) & (a == 0))
    def _init():
        out_ref[0, 0] = jnp.float32(0.0)

    out_ref[0, 0] += partial


def kernel(pred, target):
    B = pred.shape[0]
    pred4 = pred.reshape(B, _NA, _NC + 5, _HW)
    out = pl.pallas_call(
        _body,
        grid=(B, _NA),
        in_specs=[
            pl.BlockSpec((1, 1, _NC + 5, _HW), lambda b, a: (b, a, 0, 0)),
            pl.BlockSpec((1, _NT, 5), lambda b, a: (b, 0, 0)),
        ],
        out_specs=pl.BlockSpec(
            (1, 1), lambda b, a: (0, 0), memory_space=pltpu.SMEM),
        out_shape=jax.ShapeDtypeStruct((1, 1), jnp.float32),
    )(pred4, target)
    return out[0, 0]


# speedup vs baseline: 3.4227x; 2.1033x over previous
import jax
import jax.numpy as jnp
from jax import lax
from jax.experimental import pallas as pl
from jax.experimental.pallas import tpu as pltpu

_NA = 5
_NC = 20
_H = 19
_W = 19
_NT = 50
_HW = _H * _W
_AW = (1.3221, 3.19275, 5.05587, 9.47112, 11.2364)
_AH = (1.73145, 4.00944, 8.09892, 4.84053, 10.0071)
_OBJECT_SCALE = 5.0
_NO_OBJECT_SCALE = 1.0
_SIL_THRESH = 0.6


def _iou_parts(x1, y1, w1, h1, x2, y2, w2, h2):
    mx = jnp.minimum(x1 - w1 / 2.0, x2 - w2 / 2.0)
    Mx = jnp.maximum(x1 + w1 / 2.0, x2 + w2 / 2.0)
    my = jnp.minimum(y1 - h1 / 2.0, y2 - h2 / 2.0)
    My = jnp.maximum(y1 + h1 / 2.0, y2 + h2 / 2.0)
    uw = Mx - mx
    uh = My - my
    cw = w1 + w2 - uw
    ch = h1 + h2 - uh
    mask = (cw > 0) & (ch > 0)
    inter = jnp.where(mask, cw * ch, 0.0)
    union = w1 * h1 + w2 * h2 - inter
    return inter / jnp.maximum(union, 1e-12)


def _body(pred_ref, tgt_ref, tgtT_ref, out_ref):
    b = pl.program_id(0)
    tgt = tgt_ref[0]
    tgtT = tgtT_ref[0]

    gxn_c = tgt[:, 1:2]
    valid_c = gxn_c > 0
    gx_c = gxn_c * _W
    gy_c = tgt[:, 2:3] * _H
    gw_c = tgt[:, 3:4] * _W
    gh_c = tgt[:, 4:5] * _H

    lane5 = lax.broadcasted_iota(jnp.int32, (_NT, _NA), 1)
    aw5 = jnp.zeros((_NT, _NA), jnp.float32)
    ah5 = jnp.zeros((_NT, _NA), jnp.float32)
    for k in range(_NA):
        aw5 = jnp.where(lane5 == k, jnp.float32(_AW[k]), aw5)
        ah5 = jnp.where(lane5 == k, jnp.float32(_AH[k]), ah5)
    z = jnp.float32(0.0)
    anc_iou_c = _iou_parts(z, z, gw_c, gh_c, z, z, aw5, ah5)
    m5_c = jnp.max(anc_iou_c, axis=1, keepdims=True)
    best_c = jnp.min(jnp.where(anc_iou_c == m5_c, lane5, _NA), axis=1,
                     keepdims=True)

    gi_c = jnp.clip(jnp.floor(gx_c).astype(jnp.int32), 0, _W - 1)
    gj_c = jnp.clip(jnp.floor(gy_c).astype(jnp.int32), 0, _H - 1)
    key_c = jnp.where(valid_c, best_c * _HW + gj_c * _W + gi_c, -1)

    gwm_c = jnp.where(valid_c, gw_c, 0.0)
    gl_c = gx_c - gwm_c / 2.0
    gr_c = gx_c + gwm_c / 2.0
    gt_c = gy_c - gh_c / 2.0
    gb_c = gy_c + gh_c / 2.0
    garea_c = gwm_c * gh_c

    tcls_r = tgtT[0:1, :]
    gx_r = tgtT[1:2, :] * _W
    gy_r = tgtT[2:3, :] * _H
    gw_r = tgtT[3:4, :] * _W
    gh_r = tgtT[4:5, :] * _H

    sub5 = lax.broadcasted_iota(jnp.int32, (_NA, _NT), 0)
    aw5r = jnp.zeros((_NA, _NT), jnp.float32)
    ah5r = jnp.zeros((_NA, _NT), jnp.float32)
    for k in range(_NA):
        aw5r = jnp.where(sub5 == k, jnp.float32(_AW[k]), aw5r)
        ah5r = jnp.where(sub5 == k, jnp.float32(_AH[k]), ah5r)
    anc_iou_r = _iou_parts(z, z, gw_r, gh_r, z, z, aw5r, ah5r)
    m5_r = jnp.max(anc_iou_r, axis=0, keepdims=True)
    best_r = jnp.min(jnp.where(anc_iou_r == m5_r, sub5, _NA), axis=0,
                     keepdims=True)

    gif_r = jnp.clip(jnp.floor(gx_r), 0.0, float(_W - 1))
    gjf_r = jnp.clip(jnp.floor(gy_r), 0.0, float(_H - 1))
    aw_b = jnp.zeros_like(gw_r)
    ah_b = jnp.zeros_like(gh_r)
    for k in range(_NA):
        aw_b = jnp.where(best_r == k, jnp.float32(_AW[k]), aw_b)
        ah_b = jnp.where(best_r == k, jnp.float32(_AH[k]), ah_b)
    tbx_r = gx_r - gif_r
    tby_r = gy_r - gjf_r
    tbw_r = jnp.log(jnp.maximum(gw_r, 1e-12) / aw_b)
    tbh_r = jnp.log(jnp.maximum(gh_r, 1e-12) / ah_b)
    V = jnp.concatenate([tbx_r, tby_r, tbw_r, tbh_r, tcls_r], axis=0)

    pos = lax.broadcasted_iota(jnp.int32, (1, _HW), 1)
    ix = (pos % _W).astype(jnp.float32)
    jy = (pos // _W).astype(jnp.float32)
    tio1 = lax.broadcasted_iota(jnp.int32, (_NT, _HW), 0) + 1

    partial = jnp.float32(0.0)
    for a in range(_NA):
        p = pred_ref[0, a]
        sigx = jax.nn.sigmoid(p[0:1, :])
        sigy = jax.nn.sigmoid(p[1:2, :])
        tw = p[2:3, :]
        th = p[3:4, :]
        conf = jax.nn.sigmoid(p[4:5, :])
        cls = p[5:25, :]

        bx = sigx + ix
        by = sigy + jy
        bw = jnp.exp(tw) * jnp.float32(_AW[a])
        bh = jnp.exp(th) * jnp.float32(_AH[a])
        bl = bx - bw / 2.0
        br = bx + bw / 2.0
        bt = by - bh / 2.0
        bb = by + bh / 2.0
        barea = bw * bh

        uw = jnp.maximum(gr_c, br) - jnp.minimum(gl_c, bl)
        uh = jnp.maximum(gb_c, bb) - jnp.minimum(gt_c, bt)
        cw = (gwm_c + bw) - uw
        ch = (gh_c + bh) - uh
        inter = jnp.maximum(cw, 0.0) * jnp.maximum(ch, 0.0)
        union = (garea_c + barea) - inter
        ious = inter / jnp.maximum(union, 1e-12)

        max_iou = jnp.max(ious, axis=0, keepdims=True)
        conf_base = jnp.where(max_iou > _SIL_THRESH, 0.0, _NO_OBJECT_SCALE)

        keyrow = a * _HW + pos
        rmax = jnp.max(jnp.where(key_c == keyrow, tio1, 0), axis=0,
                       keepdims=True)
        owned = rmax > 0
        O = (tio1 == rmax).astype(jnp.float32)

        selv = lax.dot_general(V, O, (((1,), (0,)), ((), ())),
                               preferred_element_type=jnp.float32)
        tb0 = jnp.where(owned, selv[0:1, :], 0.5)
        tb1 = jnp.where(owned, selv[1:2, :], 0.5)
        tb2 = jnp.where(owned, selv[2:3, :], 0.0)
        tb3 = jnp.where(owned, selv[3:4, :], 0.0)
        tconf = jnp.where(owned, jnp.sum(O * ious, axis=0, keepdims=True), 0.0)
        kc = jnp.clip(selv[4:5, :].astype(jnp.int32), 0, _NC - 1)

        lcoord = (jnp.sum((sigx - tb0) ** 2) + jnp.sum((sigy - tb1) ** 2)
                  + jnp.sum((tw - tb2) ** 2) + jnp.sum((th - tb3) ** 2))

        cm = jnp.where(owned, _OBJECT_SCALE, conf_base)
        lconf = jnp.sum(cm * (conf - tconf) ** 2)

        mC = jnp.max(cls, axis=0, keepdims=True)
        lse = jnp.log(jnp.sum(jnp.exp(cls - mC), axis=0, keepdims=True)) + mC
        kio = lax.broadcasted_iota(jnp.int32, (_NC, _HW), 0)
        logit_sel = jnp.sum(jnp.where(kio == kc, cls, 0.0), axis=0,
                            keepdims=True)
        lcls = jnp.sum(jnp.where(owned, lse - logit_sel, 0.0))

        partial = partial + 0.5 * (lcoord + lconf) + lcls

    @pl.when(b == 0)
    def _init():
        out_ref[0, 0] = jnp.float32(0.0)

    out_ref[0, 0] += partial


def kernel(pred, target):
    B = pred.shape[0]
    pred4 = pred.reshape(B, _NA, _NC + 5, _HW)
    targetT = target.transpose(0, 2, 1)
    out = pl.pallas_call(
        _body,
        grid=(B,),
        in_specs=[
            pl.BlockSpec((1, _NA, _NC + 5, _HW), lambda b: (b, 0, 0, 0)),
            pl.BlockSpec((1, _NT, 5), lambda b: (b, 0, 0)),
            pl.BlockSpec((1, 5, _NT), lambda b: (b, 0, 0)),
        ],
        out_specs=pl.BlockSpec(
            (1, 1), lambda b: (0, 0), memory_space=pltpu.SMEM),
        out_shape=jax.ShapeDtypeStruct((1, 1), jnp.float32),
    )(pred4, target, targetT)
    return out[0, 0]
